# baseline (speedup 1.0000x reference)
import functools

import jax
import jax.numpy as jnp
from jax import lax
from jax.experimental import pallas as pl
from jax.experimental.pallas import tpu as pltpu
from jax.experimental.pallas import tpu_sc as plsc

_B = 2
_N0 = 4096
_N1 = 1024
_K = 16
_NC, _NS = 2, 16
_NW = _NC * _NS
_CH = 128


def _topk_body(pwd_ref, out_ref, *, n, rblk):
    d = pwd_ref[0]
    inf = jnp.float32(jnp.inf)
    col = lax.broadcasted_iota(jnp.int32, (rblk, n), 1)
    lane = lax.broadcasted_iota(jnp.int32, (rblk, _K), 1)
    loc = jnp.zeros((rblk, _K), jnp.int32)
    dprev = jnp.full((rblk, 1), -jnp.inf, jnp.float32)
    iprev = jnp.full((rblk, 1), -1, jnp.int32)
    for j in range(_K):
        later = (d > dprev) | ((d == dprev) & (col > iprev))
        dm = jnp.where(later, d, inf)
        m1 = jnp.min(dm, axis=1, keepdims=True)
        cand = jnp.where(dm == m1, col, n)
        sel = jnp.min(cand, axis=1, keepdims=True)
        loc = jnp.where(lane == j, sel, loc)
        dprev, iprev = m1, sel
    out_ref[0] = loc


def _topk16(pwd, n):
    rblk = 256
    return pl.pallas_call(
        functools.partial(_topk_body, n=n, rblk=rblk),
        grid=(_B, n // rblk),
        in_specs=[pl.BlockSpec((1, rblk, n), lambda b, i: (b, i, 0))],
        out_specs=pl.BlockSpec((1, rblk, _K), lambda b, i: (b, i, 0)),
        out_shape=jax.ShapeDtypeStruct((_B, n, _K), jnp.int32),
    )(pwd)


def _gather_rows(table, idx):
    m_rows, c = idx.shape[0], table.shape[1]
    rows_k = m_rows // _K
    per_w = m_rows // _NW
    nch = per_w // _CH

    @functools.partial(
        pl.kernel,
        out_type=jax.ShapeDtypeStruct((_K, rows_k, c), jnp.float32),
        mesh=plsc.VectorSubcoreMesh(core_axis_name="c", subcore_axis_name="s",
                                    num_cores=_NC, num_subcores=_NS),
        scratch_types=[
            pltpu.VMEM((_CH,), jnp.int32),
            pltpu.VMEM((_CH, c), jnp.float32),
            pltpu.SemaphoreType.DMA,
        ],
        compiler_params=pltpu.CompilerParams(use_tc_tiling_on_sc=False),
    )
    def scg(table_hbm, idx_hbm, out_hbm, idx_v, rows_v, sem):
        wid = lax.axis_index("s") * _NC + lax.axis_index("c")
        base = pl.multiple_of(wid * per_w, _CH)

        def chunk(i, carry):
            off = pl.multiple_of(base + i * _CH, _CH)
            pltpu.sync_copy(idx_hbm.at[pl.ds(off, _CH)], idx_v)
            pltpu.async_copy(table_hbm.at[idx_v], rows_v, sem).wait()
            k_id = off // rows_k
            r0 = pl.multiple_of(off - k_id * rows_k, _CH)
            pltpu.sync_copy(rows_v, out_hbm.at[k_id, pl.ds(r0, _CH)])
            return carry

        lax.fori_loop(0, nch, chunk, 0)

    return scg(table, idx)


def _stack_idx(knn, stride):
    g = knn + (jnp.arange(_B, dtype=jnp.int32) * stride)[:, None, None]
    return jnp.transpose(g, (2, 0, 1)).reshape(-1)


def _bnp(h, w, b):
    mu = jnp.mean(h, axis=1, keepdims=True)
    var = jnp.mean((h - mu) * (h - mu), axis=1, keepdims=True)
    return (h - mu) / jnp.sqrt(var + 1e-5) * w + b


def _dot(a, b):
    return jnp.dot(a, b, preferred_element_type=jnp.float32)


def _dot_tl(a, b):
    return lax.dot_general(a, b, (((0,), (0,)), ((), ())),
                           preferred_element_type=jnp.float32)


def _spse_acc(gx_ref, xyz, mt, inv):
    n = xyz.shape[1]
    acc = jnp.zeros((mt.shape[0], n), jnp.float32)
    for j in range(_K):
        rel = (gx_ref[j] - xyz) * inv
        rx, ry, rz = rel[0:1], rel[1:2], rel[2:3]
        feats = jnp.concatenate(
            [rx, ry, rz, rx * rx, ry * ry, rz * rz, rx * ry, rx * rz,
             ry * rz, jnp.abs(rx), jnp.abs(ry), jnp.abs(rz)], axis=0)
        resp = _dot(mt, feats)
        acc = acc + resp * resp
    return acc * (1.0 / _K)


def _mlp_res(x, w1t, b1, w2t, bw, bb):
    t = jax.nn.gelu(_dot(w1t, x) + b1)
    t = _dot(w2t, t)
    return x + _bnp(t, bw, bb)


def _stage0_head_body(gx_ref, xyz_ref, mt_ref, npbw, npbb, npw1t, npb1, npw2t,
                      nbw, nbb, mw1t, mb1, mw2t, mbw, mbb, proj_ref,
                      x_out, y_out):
    nbr = jnp.sqrt(_spse_acc(gx_ref, xyz_ref[...], mt_ref[...], 1.0) + 1e-12)
    h = _bnp(nbr, npbw[...], npbb[...])
    h = jax.nn.gelu(_dot(npw1t[...], h) + npb1[...])
    h = _dot(npw2t[...], h)
    h0 = _bnp(h, nbw[...], nbb[...])
    x = _mlp_res(h0, mw1t[...], mb1[...], mw2t[...], mbw[...], mbb[...])
    x_out[...] = x
    y_out[...] = _dot_tl(x, proj_ref[...])


def _wmean_body(g_ref, gx_ref, xyz_ref, coor_ref, cn_ref, s2_ref, rep_ref,
                out_ref, *, inv, rb, c):
    xyz = xyz_ref[...]
    coor = coor_ref[...]
    cn = cn_ref[...]
    s2 = s2_ref[...]
    rep = rep_ref[...]
    acc = jnp.zeros((rb, c), jnp.float32)
    for j in range(_K):
        rel = (gx_ref[j] - xyz) * inv
        rn = jnp.sum(rel * rel, axis=0, keepdims=True)
        dd = rn + cn - 2.0 * _dot(coor, rel)
        w = jnp.exp(-s2 * dd)
        acc = acc + _dot(jnp.transpose(w), rep) * g_ref[j]
    out_ref[...] = acc * (1.0 / _K)


def _wmean(g, gxt, xyzt, p, inv, c):
    r = xyzt.shape[1]
    d4 = c // 4
    rb = 512
    coor = p['coor'].reshape(d4, 3)
    cn = jnp.sum(coor * coor, axis=1)[:, None]
    s2 = (p['scale'] ** 2)[:, None]
    rep = jnp.repeat(jnp.eye(d4, dtype=jnp.float32), 4, axis=1)
    return pl.pallas_call(
        functools.partial(_wmean_body, inv=inv, rb=rb, c=c),
        grid=(r // rb,),
        in_specs=[
            pl.BlockSpec((_K, rb, c), lambda i: (0, i, 0)),
            pl.BlockSpec((_K, 3, rb), lambda i: (0, 0, i)),
            pl.BlockSpec((3, rb), lambda i: (0, i)),
            pl.BlockSpec((d4, 3), lambda i: (0, 0)),
            pl.BlockSpec((d4, 1), lambda i: (0, 0)),
            pl.BlockSpec((d4, 1), lambda i: (0, 0)),
            pl.BlockSpec((d4, c), lambda i: (0, 0)),
        ],
        out_specs=pl.BlockSpec((rb, c), lambda i: (i, 0)),
        out_shape=jax.ShapeDtypeStruct((r, c), jnp.float32),
    )(g, gxt, xyzt, coor, cn, s2, rep)


def _bn_add_proj_body(x_ref, s_ref, bw, bb, proj_ref, x_out, y_out):
    st = jnp.transpose(s_ref[...])
    x = x_ref[...] + _bnp(st, bw[...], bb[...])
    x_out[...] = x
    y_out[...] = _dot_tl(x, proj_ref[...])


def _tail0_body(x_ref, s_ref, lbw, lbb, mw1t, mb1, mw2t, mbw, mbb,
                lproj, skwt, skbw, skbb, ylfp_out, skip_out):
    st = jnp.transpose(s_ref[...])
    x = x_ref[...] + _bnp(st, lbw[...], lbb[...])
    x = _mlp_res(x, mw1t[...], mb1[...], mw2t[...], mbw[...], mbb[...])
    ylfp_out[...] = _dot_tl(x, lproj[...])
    ysk = _bnp(_dot(skwt[...], x), skbw[...], skbb[...])
    skip_out[...] = jnp.concatenate(
        [ysk[:, 0:_N1], ysk[:, _N0:_N0 + _N1]], axis=1)


def _stage1_head_body(slf_ref, lfbw, lfbb, skip_ref, gx_ref, xyzs_ref, mt_ref,
                      npbw, npbb, npw1t, npb1, npw2t, nbw, nbb,
                      mw1t, mb1, mw2t, mbw, mbb, proj_ref, x_out, y_out):
    lf_full = _bnp(jnp.transpose(slf_ref[...]), lfbw[...], lfbb[...])
    lf = jnp.concatenate(
        [lf_full[:, 0:_N1], lf_full[:, _N0:_N0 + _N1]], axis=1)
    x = skip_ref[...] + lf
    nbr = jnp.sqrt(_spse_acc(gx_ref, xyzs_ref[...], mt_ref[...], 0.5) + 1e-12)
    h = _bnp(nbr, npbw[...], npbb[...])
    h = jax.nn.gelu(_dot(npw1t[...], h) + npb1[...])
    h = _dot(npw2t[...], h)
    x = _bnp(h, nbw[...], nbb[...]) + x
    x = _mlp_res(x, mw1t[...], mb1[...], mw2t[...], mbw[...], mbb[...])
    x_out[...] = x
    y_out[...] = _dot_tl(x, proj_ref[...])


def _tail1_body(x_ref, s_ref, lbw, lbb, mw1t, mb1, mw2t, mbw, mbb, out_ref):
    st = jnp.transpose(s_ref[...])
    x = x_ref[...] + _bnp(st, lbw[...], lbb[...])
    out_ref[...] = _mlp_res(x, mw1t[...], mb1[...], mw2t[...], mbw[...],
                            mbb[...])


def _call(body, out_shapes, *args):
    return pl.pallas_call(body, out_shape=out_shapes)(*args)


def _f32(r, c):
    return jax.ShapeDtypeStruct((r, c), jnp.float32)


def _mlp_args(p):
    return (p['w1'].T, p['b1'][:, None], p['w2'].T,
            p['bn_w'][:, None], p['bn_b'][:, None])


def kernel(x, xyz, prev_knn, pwd, params):
    del x, prev_knn
    s0, s1, m = params['s0'], params['s1'], params['spse_m']
    blk0, blk1 = s0['blk'], s1['blk']
    bn0, bn1 = _B * _N0, _B * _N1
    xyzf = xyz.reshape(bn0, 3)
    xyzt = xyzf.T
    xyzp = jnp.pad(xyzf, ((0, 0), (0, 5)))

    knn0 = _topk16(pwd, _N0)
    idx0 = _stack_idx(knn0, _N0)
    gxyz0 = jnp.transpose(_gather_rows(xyzp, idx0)[:, :, :3], (0, 2, 1))

    x0, y0 = _call(
        _stage0_head_body, [_f32(96, bn0), _f32(bn0, 96)],
        gxyz0, xyzt, m.T,
        s0['np_bn_w'][:, None], s0['np_bn_b'][:, None], s0['np_w1'].T,
        s0['np_b1'][:, None], s0['np_w2'].T, s0['nbr_bn_w'][:, None],
        s0['nbr_bn_b'][:, None], *_mlp_args(blk0['mlp0']),
        blk0['lfp0']['proj'])

    gy0 = _gather_rows(y0, idx0)
    sm0 = _wmean(gy0, gxyz0, xyzt, blk0['lfp0'], 1.0, 96)
    x1, y1 = _call(
        _bn_add_proj_body, [_f32(96, bn0), _f32(bn0, 96)],
        x0, sm0, blk0['lfp0']['bn_w'][:, None], blk0['lfp0']['bn_b'][:, None],
        blk0['lfp1']['proj'])

    gy1 = _gather_rows(y1, idx0)
    sm1 = _wmean(gy1, gxyz0, xyzt, blk0['lfp1'], 1.0, 96)
    ylfp, skip = _call(
        _tail0_body, [_f32(bn0, 192), _f32(192, bn1)],
        x1, sm1, blk0['lfp1']['bn_w'][:, None], blk0['lfp1']['bn_b'][:, None],
        *_mlp_args(blk0['mlps0']),
        s1['lfp']['proj'], s1['skip_w'].T, s1['skip_bn_w'][:, None],
        s1['skip_bn_b'][:, None])

    glf = _gather_rows(ylfp, idx0)
    slf = _wmean(glf, gxyz0, xyzt, s1['lfp'], 1.0, 192)

    knn1 = _topk16(pwd, _N1)
    idx1x = _stack_idx(knn1, _N0)
    idx1f = _stack_idx(knn1, _N1)
    gxyz1 = jnp.transpose(_gather_rows(xyzp, idx1x)[:, :, :3], (0, 2, 1))
    xyzst = jnp.concatenate([xyzt[:, 0:_N1], xyzt[:, _N0:_N0 + _N1]], axis=1)

    x2, y2 = _call(
        _stage1_head_body, [_f32(192, bn1), _f32(bn1, 192)],
        slf, s1['lfp']['bn_w'][:, None], s1['lfp']['bn_b'][:, None],
        skip, gxyz1, xyzst, m.T,
        s1['np_bn_w'][:, None], s1['np_bn_b'][:, None], s1['np_w1'].T,
        s1['np_b1'][:, None], s1['np_w2'].T, s1['nbr_bn_w'][:, None],
        s1['nbr_bn_b'][:, None], *_mlp_args(blk1['mlp0']),
        blk1['lfp0']['proj'])

    gy2 = _gather_rows(y2, idx1f)
    sm2 = _wmean(gy2, gxyz1, xyzst, blk1['lfp0'], 0.5, 192)
    x3, y3 = _call(
        _bn_add_proj_body, [_f32(192, bn1), _f32(bn1, 192)],
        x2, sm2, blk1['lfp0']['bn_w'][:, None], blk1['lfp0']['bn_b'][:, None],
        blk1['lfp1']['proj'])

    gy3 = _gather_rows(y3, idx1f)
    sm3 = _wmean(gy3, gxyz1, xyzst, blk1['lfp1'], 0.5, 192)
    out = _call(
        _tail1_body, [_f32(192, bn1)],
        x3, sm3, blk1['lfp1']['bn_w'][:, None], blk1['lfp1']['bn_b'][:, None],
        *_mlp_args(blk1['mlps0']))[0]

    return out.T.reshape(_B, _N1, 192)


# speedup vs baseline: 1.1920x; 1.1920x over previous
import functools

import jax
import jax.numpy as jnp
from jax import lax
from jax.experimental import pallas as pl
from jax.experimental.pallas import tpu as pltpu
from jax.experimental.pallas import tpu_sc as plsc

_B = 2
_N0 = 4096
_N1 = 1024
_K = 16
_NC, _NS = 2, 16
_NW = _NC * _NS
_CH = 128


def _topk_body(pwd_ref, out_ref, *, n, rblk):
    d = pwd_ref[0]
    col = lax.broadcasted_iota(jnp.int32, (rblk, n), 1)
    lane = lax.broadcasted_iota(jnp.int32, (rblk, _K), 1)
    loc = jnp.zeros((rblk, _K), jnp.int32)
    for j in range(_K):
        mn = jnp.min(d, axis=1, keepdims=True)
        cand = jnp.where(d == mn, col, n)
        sel = jnp.min(cand, axis=1, keepdims=True)
        loc = jnp.where(lane == j, sel, loc)
        d = jnp.where(col == sel, jnp.float32(jnp.inf), d)
    out_ref[0] = loc


def _topk16(pwd, n):
    rblk = 256
    return pl.pallas_call(
        functools.partial(_topk_body, n=n, rblk=rblk),
        grid=(_B, n // rblk),
        in_specs=[pl.BlockSpec((1, rblk, n), lambda b, i: (b, i, 0))],
        out_specs=pl.BlockSpec((1, rblk, _K), lambda b, i: (b, i, 0)),
        out_shape=jax.ShapeDtypeStruct((_B, n, _K), jnp.int32),
    )(pwd)


def _gather_rows(table, idx):
    m_rows, c = idx.shape[0], table.shape[1]
    rows_k = m_rows // _K
    per_w = m_rows // _NW
    nch = per_w // _CH

    @functools.partial(
        pl.kernel,
        out_type=jax.ShapeDtypeStruct((_K, rows_k, c), jnp.float32),
        mesh=plsc.VectorSubcoreMesh(core_axis_name="c", subcore_axis_name="s",
                                    num_cores=_NC, num_subcores=_NS),
        scratch_types=[
            pltpu.VMEM((_CH,), jnp.int32),
            pltpu.VMEM((_CH, c), jnp.float32),
            pltpu.SemaphoreType.DMA,
        ],
        compiler_params=pltpu.CompilerParams(use_tc_tiling_on_sc=False),
    )
    def scg(table_hbm, idx_hbm, out_hbm, idx_v, rows_v, sem):
        wid = lax.axis_index("s") * _NC + lax.axis_index("c")
        base = pl.multiple_of(wid * per_w, _CH)

        def chunk(i, carry):
            off = pl.multiple_of(base + i * _CH, _CH)
            pltpu.sync_copy(idx_hbm.at[pl.ds(off, _CH)], idx_v)
            pltpu.async_copy(table_hbm.at[idx_v], rows_v, sem).wait()
            k_id = off // rows_k
            r0 = pl.multiple_of(off - k_id * rows_k, _CH)
            pltpu.sync_copy(rows_v, out_hbm.at[k_id, pl.ds(r0, _CH)])
            return carry

        lax.fori_loop(0, nch, chunk, 0)

    return scg(table, idx)


def _stack_idx(knn, stride):
    g = knn + (jnp.arange(_B, dtype=jnp.int32) * stride)[:, None, None]
    return jnp.transpose(g, (2, 0, 1)).reshape(-1)


def _bnp(h, w, b):
    mu = jnp.mean(h, axis=1, keepdims=True)
    var = jnp.mean((h - mu) * (h - mu), axis=1, keepdims=True)
    return (h - mu) / jnp.sqrt(var + 1e-5) * w + b


def _dot(a, b):
    return jnp.dot(a, b, preferred_element_type=jnp.float32)


def _dot_tl(a, b):
    return lax.dot_general(a, b, (((0,), (0,)), ((), ())),
                           preferred_element_type=jnp.float32)


def _spse_acc(gx_ref, xyz, mt, inv):
    n = xyz.shape[1]
    acc = jnp.zeros((mt.shape[0], n), jnp.float32)
    for j in range(_K):
        rel = (gx_ref[j] - xyz) * inv
        rx, ry, rz = rel[0:1], rel[1:2], rel[2:3]
        feats = jnp.concatenate(
            [rx, ry, rz, rx * rx, ry * ry, rz * rz, rx * ry, rx * rz,
             ry * rz, jnp.abs(rx), jnp.abs(ry), jnp.abs(rz)], axis=0)
        resp = _dot(mt, feats)
        acc = acc + resp * resp
    return acc * (1.0 / _K)


def _mlp_res(x, w1t, b1, w2t, bw, bb):
    t = jax.nn.gelu(_dot(w1t, x) + b1)
    t = _dot(w2t, t)
    return x + _bnp(t, bw, bb)


def _stage0_head_body(gx_ref, xyz_ref, mt_ref, npbw, npbb, npw1t, npb1, npw2t,
                      nbw, nbb, mw1t, mb1, mw2t, mbw, mbb, proj_ref,
                      x_out, y_out):
    nbr = jnp.sqrt(_spse_acc(gx_ref, xyz_ref[...], mt_ref[...], 1.0) + 1e-12)
    h = _bnp(nbr, npbw[...], npbb[...])
    h = jax.nn.gelu(_dot(npw1t[...], h) + npb1[...])
    h = _dot(npw2t[...], h)
    h0 = _bnp(h, nbw[...], nbb[...])
    x = _mlp_res(h0, mw1t[...], mb1[...], mw2t[...], mbw[...], mbb[...])
    x_out[...] = x
    y_out[...] = _dot_tl(x, proj_ref[...])


def _wmean_body(g_ref, gx_ref, xyz_ref, coor_ref, cn_ref, s2_ref, rep_ref,
                out_ref, *, inv, rb, c):
    xyz = xyz_ref[...]
    coor = coor_ref[...]
    cn = cn_ref[...]
    s2 = s2_ref[...]
    rep = rep_ref[...]
    acc = jnp.zeros((rb, c), jnp.float32)
    for j in range(_K):
        rel = (gx_ref[j] - xyz) * inv
        rn = jnp.sum(rel * rel, axis=0, keepdims=True)
        dd = rn + cn - 2.0 * _dot(coor, rel)
        w = jnp.exp(-s2 * dd)
        acc = acc + _dot(jnp.transpose(w), rep) * g_ref[j][:, 0:c]
    out_ref[...] = acc * (1.0 / _K)


def _wmean(g, gxt, xyzt, p, inv, c):
    r = xyzt.shape[1]
    cp = g.shape[2]
    d4 = c // 4
    rb = 512
    coor = p['coor'].reshape(d4, 3)
    cn = jnp.sum(coor * coor, axis=1)[:, None]
    s2 = (p['scale'] ** 2)[:, None]
    rep = jnp.repeat(jnp.eye(d4, dtype=jnp.float32), 4, axis=1)
    return pl.pallas_call(
        functools.partial(_wmean_body, inv=inv, rb=rb, c=c),
        grid=(r // rb,),
        in_specs=[
            pl.BlockSpec((_K, rb, cp), lambda i: (0, i, 0)),
            pl.BlockSpec((_K, 3, rb), lambda i: (0, 0, i)),
            pl.BlockSpec((3, rb), lambda i: (0, i)),
            pl.BlockSpec((d4, 3), lambda i: (0, 0)),
            pl.BlockSpec((d4, 1), lambda i: (0, 0)),
            pl.BlockSpec((d4, 1), lambda i: (0, 0)),
            pl.BlockSpec((d4, c), lambda i: (0, 0)),
        ],
        out_specs=pl.BlockSpec((rb, c), lambda i: (i, 0)),
        out_shape=jax.ShapeDtypeStruct((r, c), jnp.float32),
    )(g, gxt, xyzt, coor, cn, s2, rep)


def _bn_add_proj_body(x_ref, s_ref, bw, bb, proj_ref, x_out, y_out):
    st = jnp.transpose(s_ref[...])
    x = x_ref[...] + _bnp(st, bw[...], bb[...])
    x_out[...] = x
    y_out[...] = _dot_tl(x, proj_ref[...])


def _tail0_body(x_ref, s_ref, lbw, lbb, mw1t, mb1, mw2t, mbw, mbb,
                lproj, skwt, skbw, skbb, ylfp_out, skip_out):
    st = jnp.transpose(s_ref[...])
    x = x_ref[...] + _bnp(st, lbw[...], lbb[...])
    x = _mlp_res(x, mw1t[...], mb1[...], mw2t[...], mbw[...], mbb[...])
    ylfp_out[...] = _dot_tl(x, lproj[...])
    ysk = _bnp(_dot(skwt[...], x), skbw[...], skbb[...])
    skip_out[...] = jnp.concatenate(
        [ysk[:, 0:_N1], ysk[:, _N0:_N0 + _N1]], axis=1)


def _stage1_head_body(slf_ref, lfbw, lfbb, skip_ref, gx_ref, xyzs_ref, mt_ref,
                      npbw, npbb, npw1t, npb1, npw2t, nbw, nbb,
                      mw1t, mb1, mw2t, mbw, mbb, proj_ref, x_out, y_out):
    lf_full = _bnp(jnp.transpose(slf_ref[...]), lfbw[...], lfbb[...])
    lf = jnp.concatenate(
        [lf_full[:, 0:_N1], lf_full[:, _N0:_N0 + _N1]], axis=1)
    x = skip_ref[...] + lf
    nbr = jnp.sqrt(_spse_acc(gx_ref, xyzs_ref[...], mt_ref[...], 0.5) + 1e-12)
    h = _bnp(nbr, npbw[...], npbb[...])
    h = jax.nn.gelu(_dot(npw1t[...], h) + npb1[...])
    h = _dot(npw2t[...], h)
    x = _bnp(h, nbw[...], nbb[...]) + x
    x = _mlp_res(x, mw1t[...], mb1[...], mw2t[...], mbw[...], mbb[...])
    x_out[...] = x
    y_out[...] = _dot_tl(x, proj_ref[...])


def _tail1_body(x_ref, s_ref, lbw, lbb, mw1t, mb1, mw2t, mbw, mbb, out_ref):
    st = jnp.transpose(s_ref[...])
    x = x_ref[...] + _bnp(st, lbw[...], lbb[...])
    out_ref[...] = _mlp_res(x, mw1t[...], mb1[...], mw2t[...], mbw[...],
                            mbb[...])


def _call(body, out_shapes, *args):
    return pl.pallas_call(body, out_shape=out_shapes)(*args)


def _f32(r, c):
    return jax.ShapeDtypeStruct((r, c), jnp.float32)


def _pad_cols(w, to):
    return jnp.pad(w, ((0, 0), (0, to - w.shape[1])))


def _mlp_args(p):
    return (p['w1'].T, p['b1'][:, None], p['w2'].T,
            p['bn_w'][:, None], p['bn_b'][:, None])


def kernel(x, xyz, prev_knn, pwd, params):
    del x, prev_knn
    s0, s1, m = params['s0'], params['s1'], params['spse_m']
    blk0, blk1 = s0['blk'], s1['blk']
    bn0, bn1 = _B * _N0, _B * _N1
    xyzf = xyz.reshape(bn0, 3)
    xyzt = xyzf.T
    xyzp = jnp.pad(xyzf, ((0, 0), (0, 5)))

    knn0 = _topk16(pwd, _N0)
    idx0 = _stack_idx(knn0, _N0)
    gxyz0 = jnp.transpose(_gather_rows(xyzp, idx0)[:, :, :3], (0, 2, 1))

    x0, y0 = _call(
        _stage0_head_body, [_f32(96, bn0), _f32(bn0, 128)],
        gxyz0, xyzt, m.T,
        s0['np_bn_w'][:, None], s0['np_bn_b'][:, None], s0['np_w1'].T,
        s0['np_b1'][:, None], s0['np_w2'].T, s0['nbr_bn_w'][:, None],
        s0['nbr_bn_b'][:, None], *_mlp_args(blk0['mlp0']),
        _pad_cols(blk0['lfp0']['proj'], 128))

    gy0 = _gather_rows(y0, idx0)
    sm0 = _wmean(gy0, gxyz0, xyzt, blk0['lfp0'], 1.0, 96)
    x1, y1 = _call(
        _bn_add_proj_body, [_f32(96, bn0), _f32(bn0, 128)],
        x0, sm0, blk0['lfp0']['bn_w'][:, None], blk0['lfp0']['bn_b'][:, None],
        _pad_cols(blk0['lfp1']['proj'], 128))

    gy1 = _gather_rows(y1, idx0)
    sm1 = _wmean(gy1, gxyz0, xyzt, blk0['lfp1'], 1.0, 96)
    ylfp, skip = _call(
        _tail0_body, [_f32(bn0, 256), _f32(192, bn1)],
        x1, sm1, blk0['lfp1']['bn_w'][:, None], blk0['lfp1']['bn_b'][:, None],
        *_mlp_args(blk0['mlps0']),
        _pad_cols(s1['lfp']['proj'], 256), s1['skip_w'].T, s1['skip_bn_w'][:, None],
        s1['skip_bn_b'][:, None])

    glf = _gather_rows(ylfp, idx0)
    slf = _wmean(glf, gxyz0, xyzt, s1['lfp'], 1.0, 192)

    knn1 = _topk16(pwd, _N1)
    idx1x = _stack_idx(knn1, _N0)
    idx1f = _stack_idx(knn1, _N1)
    gxyz1 = jnp.transpose(_gather_rows(xyzp, idx1x)[:, :, :3], (0, 2, 1))
    xyzst = jnp.concatenate([xyzt[:, 0:_N1], xyzt[:, _N0:_N0 + _N1]], axis=1)

    x2, y2 = _call(
        _stage1_head_body, [_f32(192, bn1), _f32(bn1, 256)],
        slf, s1['lfp']['bn_w'][:, None], s1['lfp']['bn_b'][:, None],
        skip, gxyz1, xyzst, m.T,
        s1['np_bn_w'][:, None], s1['np_bn_b'][:, None], s1['np_w1'].T,
        s1['np_b1'][:, None], s1['np_w2'].T, s1['nbr_bn_w'][:, None],
        s1['nbr_bn_b'][:, None], *_mlp_args(blk1['mlp0']),
        _pad_cols(blk1['lfp0']['proj'], 256))

    gy2 = _gather_rows(y2, idx1f)
    sm2 = _wmean(gy2, gxyz1, xyzst, blk1['lfp0'], 0.5, 192)
    x3, y3 = _call(
        _bn_add_proj_body, [_f32(192, bn1), _f32(bn1, 256)],
        x2, sm2, blk1['lfp0']['bn_w'][:, None], blk1['lfp0']['bn_b'][:, None],
        _pad_cols(blk1['lfp1']['proj'], 256))

    gy3 = _gather_rows(y3, idx1f)
    sm3 = _wmean(gy3, gxyz1, xyzst, blk1['lfp1'], 0.5, 192)
    out = _call(
        _tail1_body, [_f32(192, bn1)],
        x3, sm3, blk1['lfp1']['bn_w'][:, None], blk1['lfp1']['bn_b'][:, None],
        *_mlp_args(blk1['mlps0']))[0]

    return out.T.reshape(_B, _N1, 192)


# speedup vs baseline: 1.3055x; 1.0952x over previous
import functools

import jax
import jax.numpy as jnp
from jax import lax
from jax.experimental import pallas as pl
from jax.experimental.pallas import tpu as pltpu
from jax.experimental.pallas import tpu_sc as plsc

_B = 2
_N0 = 4096
_N1 = 1024
_K = 16
_NC, _NS = 2, 16
_NW = _NC * _NS
_CH = 128


def _topk_body(pwd_ref, out_ref, *, n, rblk):
    d = pwd_ref[0]
    col = lax.broadcasted_iota(jnp.int32, (rblk, n), 1)
    lane = lax.broadcasted_iota(jnp.int32, (rblk, _K), 1)
    loc = jnp.zeros((rblk, _K), jnp.int32)
    for j in range(_K):
        mn = jnp.min(d, axis=1, keepdims=True)
        cand = jnp.where(d == mn, col, n)
        sel = jnp.min(cand, axis=1, keepdims=True)
        loc = jnp.where(lane == j, sel, loc)
        d = jnp.where(col == sel, jnp.float32(jnp.inf), d)
    out_ref[0] = loc


def _topk16(pwd, n):
    rblk = 256
    return pl.pallas_call(
        functools.partial(_topk_body, n=n, rblk=rblk),
        grid=(_B, n // rblk),
        in_specs=[pl.BlockSpec((1, rblk, n), lambda b, i: (b, i, 0))],
        out_specs=pl.BlockSpec((1, rblk, _K), lambda b, i: (b, i, 0)),
        out_shape=jax.ShapeDtypeStruct((_B, n, _K), jnp.int32),
    )(pwd)


def _gather_rows(table, idx):
    m_rows, c = idx.shape[0], table.shape[1]
    rows_k = m_rows // _K
    per_w = m_rows // _NW
    nch = per_w // _CH

    @functools.partial(
        pl.kernel,
        out_type=jax.ShapeDtypeStruct((_K, rows_k, c), jnp.float32),
        mesh=plsc.VectorSubcoreMesh(core_axis_name="c", subcore_axis_name="s",
                                    num_cores=_NC, num_subcores=_NS),
        scratch_types=[
            pltpu.VMEM((per_w,), jnp.int32),
            pltpu.VMEM((_CH, c), jnp.float32),
            pltpu.VMEM((_CH, c), jnp.float32),
            pltpu.SemaphoreType.DMA,
            pltpu.SemaphoreType.DMA,
        ],
        compiler_params=pltpu.CompilerParams(use_tc_tiling_on_sc=False),
    )
    def scg(table_hbm, idx_hbm, out_hbm, idx_all, rows_a, rows_b, sem_a,
            sem_b):
        wid = lax.axis_index("s") * _NC + lax.axis_index("c")
        base = pl.multiple_of(wid * per_w, _CH)
        pltpu.sync_copy(idx_hbm.at[pl.ds(base, per_w)], idx_all)

        bufs = (rows_a, rows_b)
        sems = (sem_a, sem_b)

        def start(i):
            return pltpu.async_copy(
                table_hbm.at[idx_all.at[pl.ds(i * _CH, _CH)]],
                bufs[i % 2], sems[i % 2])

        waits = {0: start(i=0)}
        for i in range(nch):
            if i + 1 < nch:
                waits[i + 1] = start(i + 1)
            waits.pop(i).wait()
            off = base + i * _CH
            k_id = off // rows_k
            r0 = pl.multiple_of(off - k_id * rows_k, _CH)
            pltpu.sync_copy(bufs[i % 2], out_hbm.at[k_id, pl.ds(r0, _CH)])

    return scg(table, idx)


def _stack_idx(knn, stride):
    g = knn + (jnp.arange(_B, dtype=jnp.int32) * stride)[:, None, None]
    return jnp.transpose(g, (2, 0, 1)).reshape(-1)


def _bnp(h, w, b):
    mu = jnp.mean(h, axis=1, keepdims=True)
    var = jnp.mean((h - mu) * (h - mu), axis=1, keepdims=True)
    return (h - mu) / jnp.sqrt(var + 1e-5) * w + b


def _dot(a, b):
    return jnp.dot(a, b, preferred_element_type=jnp.float32)


def _dot_tl(a, b):
    return lax.dot_general(a, b, (((0,), (0,)), ((), ())),
                           preferred_element_type=jnp.float32)


def _spse_acc(gx_ref, xyz, mt, inv):
    n = xyz.shape[1]
    acc = jnp.zeros((mt.shape[0], n), jnp.float32)
    for j in range(_K):
        rel = (gx_ref[j] - xyz) * inv
        rx, ry, rz = rel[0:1], rel[1:2], rel[2:3]
        feats = jnp.concatenate(
            [rx, ry, rz, rx * rx, ry * ry, rz * rz, rx * ry, rx * rz,
             ry * rz, jnp.abs(rx), jnp.abs(ry), jnp.abs(rz)], axis=0)
        resp = _dot(mt, feats)
        acc = acc + resp * resp
    return acc * (1.0 / _K)


def _mlp_res(x, w1t, b1, w2t, bw, bb):
    t = jax.nn.gelu(_dot(w1t, x) + b1)
    t = _dot(w2t, t)
    return x + _bnp(t, bw, bb)


def _stage0_head_body(gx_ref, xyz_ref, mt_ref, npbw, npbb, npw1t, npb1, npw2t,
                      nbw, nbb, mw1t, mb1, mw2t, mbw, mbb, proj_ref,
                      x_out, y_out):
    nbr = jnp.sqrt(_spse_acc(gx_ref, xyz_ref[...], mt_ref[...], 1.0) + 1e-12)
    h = _bnp(nbr, npbw[...], npbb[...])
    h = jax.nn.gelu(_dot(npw1t[...], h) + npb1[...])
    h = _dot(npw2t[...], h)
    h0 = _bnp(h, nbw[...], nbb[...])
    x = _mlp_res(h0, mw1t[...], mb1[...], mw2t[...], mbw[...], mbb[...])
    x_out[...] = x
    y_out[...] = _dot_tl(x, proj_ref[...])


def _wmean_body(g_ref, gx_ref, xyz_ref, coor_ref, cn_ref, s2_ref, rep_ref,
                out_ref, *, inv, rb, c):
    xyz = xyz_ref[...]
    coor = coor_ref[...]
    cn = cn_ref[...]
    s2 = s2_ref[...]
    rep = rep_ref[...]
    acc = jnp.zeros((rb, c), jnp.float32)
    for j in range(_K):
        rel = (gx_ref[j] - xyz) * inv
        rn = jnp.sum(rel * rel, axis=0, keepdims=True)
        dd = rn + cn - 2.0 * _dot(coor, rel)
        w = jnp.exp(-s2 * dd)
        acc = acc + _dot(jnp.transpose(w), rep) * g_ref[j][:, 0:c]
    out_ref[...] = acc * (1.0 / _K)


def _wmean(g, gxt, xyzt, p, inv, c):
    r = xyzt.shape[1]
    cp = g.shape[2]
    d4 = c // 4
    rb = 512
    coor = p['coor'].reshape(d4, 3)
    cn = jnp.sum(coor * coor, axis=1)[:, None]
    s2 = (p['scale'] ** 2)[:, None]
    rep = jnp.repeat(jnp.eye(d4, dtype=jnp.float32), 4, axis=1)
    return pl.pallas_call(
        functools.partial(_wmean_body, inv=inv, rb=rb, c=c),
        grid=(r // rb,),
        in_specs=[
            pl.BlockSpec((_K, rb, cp), lambda i: (0, i, 0)),
            pl.BlockSpec((_K, 3, rb), lambda i: (0, 0, i)),
            pl.BlockSpec((3, rb), lambda i: (0, i)),
            pl.BlockSpec((d4, 3), lambda i: (0, 0)),
            pl.BlockSpec((d4, 1), lambda i: (0, 0)),
            pl.BlockSpec((d4, 1), lambda i: (0, 0)),
            pl.BlockSpec((d4, c), lambda i: (0, 0)),
        ],
        out_specs=pl.BlockSpec((rb, c), lambda i: (i, 0)),
        out_shape=jax.ShapeDtypeStruct((r, c), jnp.float32),
    )(g, gxt, xyzt, coor, cn, s2, rep)


def _bn_add_proj_body(x_ref, s_ref, bw, bb, proj_ref, x_out, y_out):
    st = jnp.transpose(s_ref[...])
    x = x_ref[...] + _bnp(st, bw[...], bb[...])
    x_out[...] = x
    y_out[...] = _dot_tl(x, proj_ref[...])


def _tail0_body(x_ref, s_ref, lbw, lbb, mw1t, mb1, mw2t, mbw, mbb,
                lproj, skwt, skbw, skbb, ylfp_out, skip_out):
    st = jnp.transpose(s_ref[...])
    x = x_ref[...] + _bnp(st, lbw[...], lbb[...])
    x = _mlp_res(x, mw1t[...], mb1[...], mw2t[...], mbw[...], mbb[...])
    ylfp_out[...] = _dot_tl(x, lproj[...])
    ysk = _bnp(_dot(skwt[...], x), skbw[...], skbb[...])
    skip_out[...] = jnp.concatenate(
        [ysk[:, 0:_N1], ysk[:, _N0:_N0 + _N1]], axis=1)


def _stage1_head_body(slf_ref, lfbw, lfbb, skip_ref, gx_ref, xyzs_ref, mt_ref,
                      npbw, npbb, npw1t, npb1, npw2t, nbw, nbb,
                      mw1t, mb1, mw2t, mbw, mbb, proj_ref, x_out, y_out):
    lf_full = _bnp(jnp.transpose(slf_ref[...]), lfbw[...], lfbb[...])
    lf = jnp.concatenate(
        [lf_full[:, 0:_N1], lf_full[:, _N0:_N0 + _N1]], axis=1)
    x = skip_ref[...] + lf
    nbr = jnp.sqrt(_spse_acc(gx_ref, xyzs_ref[...], mt_ref[...], 0.5) + 1e-12)
    h = _bnp(nbr, npbw[...], npbb[...])
    h = jax.nn.gelu(_dot(npw1t[...], h) + npb1[...])
    h = _dot(npw2t[...], h)
    x = _bnp(h, nbw[...], nbb[...]) + x
    x = _mlp_res(x, mw1t[...], mb1[...], mw2t[...], mbw[...], mbb[...])
    x_out[...] = x
    y_out[...] = _dot_tl(x, proj_ref[...])


def _tail1_body(x_ref, s_ref, lbw, lbb, mw1t, mb1, mw2t, mbw, mbb, out_ref):
    st = jnp.transpose(s_ref[...])
    x = x_ref[...] + _bnp(st, lbw[...], lbb[...])
    out_ref[...] = _mlp_res(x, mw1t[...], mb1[...], mw2t[...], mbw[...],
                            mbb[...])


def _call(body, out_shapes, *args):
    return pl.pallas_call(body, out_shape=out_shapes)(*args)


def _f32(r, c):
    return jax.ShapeDtypeStruct((r, c), jnp.float32)


def _pad_cols(w, to):
    return jnp.pad(w, ((0, 0), (0, to - w.shape[1])))


def _mlp_args(p):
    return (p['w1'].T, p['b1'][:, None], p['w2'].T,
            p['bn_w'][:, None], p['bn_b'][:, None])


def kernel(x, xyz, prev_knn, pwd, params):
    del x, prev_knn
    s0, s1, m = params['s0'], params['s1'], params['spse_m']
    blk0, blk1 = s0['blk'], s1['blk']
    bn0, bn1 = _B * _N0, _B * _N1
    xyzf = xyz.reshape(bn0, 3)
    xyzt = xyzf.T
    xyzp = jnp.pad(xyzf, ((0, 0), (0, 5)))

    knn0 = _topk16(pwd, _N0)
    idx0 = _stack_idx(knn0, _N0)
    gxyz0 = jnp.transpose(_gather_rows(xyzp, idx0)[:, :, :3], (0, 2, 1))

    x0, y0 = _call(
        _stage0_head_body, [_f32(96, bn0), _f32(bn0, 128)],
        gxyz0, xyzt, m.T,
        s0['np_bn_w'][:, None], s0['np_bn_b'][:, None], s0['np_w1'].T,
        s0['np_b1'][:, None], s0['np_w2'].T, s0['nbr_bn_w'][:, None],
        s0['nbr_bn_b'][:, None], *_mlp_args(blk0['mlp0']),
        _pad_cols(blk0['lfp0']['proj'], 128))

    gy0 = _gather_rows(y0, idx0)
    sm0 = _wmean(gy0, gxyz0, xyzt, blk0['lfp0'], 1.0, 96)
    x1, y1 = _call(
        _bn_add_proj_body, [_f32(96, bn0), _f32(bn0, 128)],
        x0, sm0, blk0['lfp0']['bn_w'][:, None], blk0['lfp0']['bn_b'][:, None],
        _pad_cols(blk0['lfp1']['proj'], 128))

    gy1 = _gather_rows(y1, idx0)
    sm1 = _wmean(gy1, gxyz0, xyzt, blk0['lfp1'], 1.0, 96)
    ylfp, skip = _call(
        _tail0_body, [_f32(bn0, 256), _f32(192, bn1)],
        x1, sm1, blk0['lfp1']['bn_w'][:, None], blk0['lfp1']['bn_b'][:, None],
        *_mlp_args(blk0['mlps0']),
        _pad_cols(s1['lfp']['proj'], 256), s1['skip_w'].T, s1['skip_bn_w'][:, None],
        s1['skip_bn_b'][:, None])

    glf = _gather_rows(ylfp, idx0)
    slf = _wmean(glf, gxyz0, xyzt, s1['lfp'], 1.0, 192)

    knn1 = _topk16(pwd, _N1)
    idx1x = _stack_idx(knn1, _N0)
    idx1f = _stack_idx(knn1, _N1)
    gxyz1 = jnp.transpose(_gather_rows(xyzp, idx1x)[:, :, :3], (0, 2, 1))
    xyzst = jnp.concatenate([xyzt[:, 0:_N1], xyzt[:, _N0:_N0 + _N1]], axis=1)

    x2, y2 = _call(
        _stage1_head_body, [_f32(192, bn1), _f32(bn1, 256)],
        slf, s1['lfp']['bn_w'][:, None], s1['lfp']['bn_b'][:, None],
        skip, gxyz1, xyzst, m.T,
        s1['np_bn_w'][:, None], s1['np_bn_b'][:, None], s1['np_w1'].T,
        s1['np_b1'][:, None], s1['np_w2'].T, s1['nbr_bn_w'][:, None],
        s1['nbr_bn_b'][:, None], *_mlp_args(blk1['mlp0']),
        _pad_cols(blk1['lfp0']['proj'], 256))

    gy2 = _gather_rows(y2, idx1f)
    sm2 = _wmean(gy2, gxyz1, xyzst, blk1['lfp0'], 0.5, 192)
    x3, y3 = _call(
        _bn_add_proj_body, [_f32(192, bn1), _f32(bn1, 256)],
        x2, sm2, blk1['lfp0']['bn_w'][:, None], blk1['lfp0']['bn_b'][:, None],
        _pad_cols(blk1['lfp1']['proj'], 256))

    gy3 = _gather_rows(y3, idx1f)
    sm3 = _wmean(gy3, gxyz1, xyzst, blk1['lfp1'], 0.5, 192)
    out = _call(
        _tail1_body, [_f32(192, bn1)],
        x3, sm3, blk1['lfp1']['bn_w'][:, None], blk1['lfp1']['bn_b'][:, None],
        *_mlp_args(blk1['mlps0']))[0]

    return out.T.reshape(_B, _N1, 192)


# speedup vs baseline: 1.5445x; 1.1831x over previous
import functools

import jax
import jax.numpy as jnp
from jax import lax
from jax.experimental import pallas as pl
from jax.experimental.pallas import tpu as pltpu
from jax.experimental.pallas import tpu_sc as plsc

_B = 2
_N0 = 4096
_N1 = 1024
_K = 16
_NC, _NS = 2, 16
_NW = _NC * _NS
_CH = 128


def _topk_body(pwd_ref, out_ref, *, n, rblk):
    d = pwd_ref[0]
    col = lax.broadcasted_iota(jnp.int32, (rblk, n), 1)
    lane = lax.broadcasted_iota(jnp.int32, (rblk, _K), 1)
    loc = jnp.zeros((rblk, _K), jnp.int32)
    for j in range(_K):
        mn = jnp.min(d, axis=1, keepdims=True)
        cand = jnp.where(d == mn, col, n)
        sel = jnp.min(cand, axis=1, keepdims=True)
        loc = jnp.where(lane == j, sel, loc)
        d = jnp.where(col == sel, jnp.float32(jnp.inf), d)
    out_ref[0] = loc


def _topk16(pwd, n):
    rblk = 256
    return pl.pallas_call(
        functools.partial(_topk_body, n=n, rblk=rblk),
        grid=(_B, n // rblk),
        in_specs=[pl.BlockSpec((1, rblk, n), lambda b, i: (b, i, 0))],
        out_specs=pl.BlockSpec((1, rblk, _K), lambda b, i: (b, i, 0)),
        out_shape=jax.ShapeDtypeStruct((_B, n, _K), jnp.int32),
    )(pwd)


def _gather_rows(table, idx):
    m_rows, c = idx.shape[0], table.shape[1]
    rows_k = m_rows // _K
    per_w = m_rows // _NW
    nch = per_w // _CH

    @functools.partial(
        pl.kernel,
        out_type=jax.ShapeDtypeStruct((_K, rows_k, c), jnp.float32),
        mesh=plsc.VectorSubcoreMesh(core_axis_name="c", subcore_axis_name="s",
                                    num_cores=_NC, num_subcores=_NS),
        scratch_types=[
            pltpu.VMEM((per_w,), jnp.int32),
            pltpu.VMEM((_CH, c), jnp.float32),
            pltpu.VMEM((_CH, c), jnp.float32),
            pltpu.SemaphoreType.DMA,
            pltpu.SemaphoreType.DMA,
        ],
        compiler_params=pltpu.CompilerParams(use_tc_tiling_on_sc=False),
    )
    def scg(table_hbm, idx_hbm, out_hbm, idx_all, rows_a, rows_b, sem_a,
            sem_b):
        wid = lax.axis_index("s") * _NC + lax.axis_index("c")
        base = pl.multiple_of(wid * per_w, _CH)
        pltpu.sync_copy(idx_hbm.at[pl.ds(base, per_w)], idx_all)

        bufs = (rows_a, rows_b)
        sems = (sem_a, sem_b)

        def start(i):
            return pltpu.async_copy(
                table_hbm.at[idx_all.at[pl.ds(i * _CH, _CH)]],
                bufs[i % 2], sems[i % 2])

        waits = {0: start(i=0)}
        for i in range(nch):
            if i + 1 < nch:
                waits[i + 1] = start(i + 1)
            waits.pop(i).wait()
            off = base + i * _CH
            k_id = off // rows_k
            r0 = pl.multiple_of(off - k_id * rows_k, _CH)
            pltpu.sync_copy(bufs[i % 2], out_hbm.at[k_id, pl.ds(r0, _CH)])

    return scg(table, idx)


def _stack_idx(knn, stride):
    g = knn + (jnp.arange(_B, dtype=jnp.int32) * stride)[:, None, None]
    return jnp.transpose(g, (2, 0, 1)).reshape(-1)


def _bnp(h, w, b):
    mu = jnp.mean(h, axis=1, keepdims=True)
    var = jnp.mean((h - mu) * (h - mu), axis=1, keepdims=True)
    return (h - mu) / jnp.sqrt(var + 1e-5) * w + b


def _dot(a, b):
    return jnp.dot(a, b, preferred_element_type=jnp.float32)


def _dot_tl(a, b):
    return lax.dot_general(a, b, (((0,), (0,)), ((), ())),
                           preferred_element_type=jnp.float32)


def _spse_acc(gx_ref, xyz, mt, inv):
    n = xyz.shape[1]
    acc = jnp.zeros((mt.shape[0], n), jnp.float32)
    for j in range(_K):
        rel = (gx_ref[j] - xyz) * inv
        rx, ry, rz = rel[0:1], rel[1:2], rel[2:3]
        feats = jnp.concatenate(
            [rx, ry, rz, rx * rx, ry * ry, rz * rz, rx * ry, rx * rz,
             ry * rz, jnp.abs(rx), jnp.abs(ry), jnp.abs(rz)], axis=0)
        resp = _dot(mt, feats)
        acc = acc + resp * resp
    return acc * (1.0 / _K)


def _mlp_res(x, w1t, b1, w2t, bw, bb):
    t = jax.nn.gelu(_dot(w1t, x) + b1)
    t = _dot(w2t, t)
    return x + _bnp(t, bw, bb)


def _stage0_head_body(gx_ref, xyz_ref, mt_ref, npbw, npbb, npw1t, npb1, npw2t,
                      nbw, nbb, mw1t, mb1, mw2t, mbw, mbb, proj_ref,
                      x_out, y_out):
    nbr = jnp.sqrt(_spse_acc(gx_ref, xyz_ref[...], mt_ref[...], 1.0) + 1e-12)
    h = _bnp(nbr, npbw[...], npbb[...])
    h = jax.nn.gelu(_dot(npw1t[...], h) + npb1[...])
    h = _dot(npw2t[...], h)
    h0 = _bnp(h, nbw[...], nbb[...])
    x = _mlp_res(h0, mw1t[...], mb1[...], mw2t[...], mbw[...], mbb[...])
    x_out[...] = x
    y_out[...] = _dot_tl(x, proj_ref[...])


def _wmean_body(g_ref, gx_ref, xyz_ref, coor_ref, cn_ref, s2_ref, rep_ref,
                out_ref, *, inv, rb, c):
    xyz = xyz_ref[...]
    coor = coor_ref[...]
    cn = cn_ref[...]
    s2 = s2_ref[...]
    rep = rep_ref[...]
    acc = jnp.zeros((rb, c), jnp.float32)
    for j in range(_K):
        rel = (gx_ref[j] - xyz) * inv
        rn = jnp.sum(rel * rel, axis=0, keepdims=True)
        dd = rn + cn - 2.0 * _dot(coor, rel)
        w = jnp.exp(-s2 * dd)
        acc = acc + _dot(jnp.transpose(w), rep) * g_ref[j][:, 0:c]
    out_ref[...] = acc * (1.0 / _K)


def _wmean(g, gxt, xyzt, p, inv, c):
    r = xyzt.shape[1]
    cp = g.shape[2]
    d4 = c // 4
    rb = 512
    coor = p['coor'].reshape(d4, 3)
    cn = jnp.sum(coor * coor, axis=1)[:, None]
    s2 = (p['scale'] ** 2)[:, None]
    rep = jnp.repeat(jnp.eye(d4, dtype=jnp.float32), 4, axis=1)
    return pl.pallas_call(
        functools.partial(_wmean_body, inv=inv, rb=rb, c=c),
        grid=(r // rb,),
        in_specs=[
            pl.BlockSpec((_K, rb, cp), lambda i: (0, i, 0)),
            pl.BlockSpec((_K, 3, rb), lambda i: (0, 0, i)),
            pl.BlockSpec((3, rb), lambda i: (0, i)),
            pl.BlockSpec((d4, 3), lambda i: (0, 0)),
            pl.BlockSpec((d4, 1), lambda i: (0, 0)),
            pl.BlockSpec((d4, 1), lambda i: (0, 0)),
            pl.BlockSpec((d4, c), lambda i: (0, 0)),
        ],
        out_specs=pl.BlockSpec((rb, c), lambda i: (i, 0)),
        out_shape=jax.ShapeDtypeStruct((r, c), jnp.float32),
    )(g, gxt, xyzt, coor, cn, s2, rep)


def _bn_add_proj_body(x_ref, s_ref, bw, bb, proj_ref, x_out, y_out):
    st = jnp.transpose(s_ref[...])
    x = x_ref[...] + _bnp(st, bw[...], bb[...])
    x_out[...] = x
    y_out[...] = _dot_tl(x, proj_ref[...])


def _tail0_body(x_ref, s_ref, lbw, lbb, mw1t, mb1, mw2t, mbw, mbb,
                lpa, lpb, skwt, skbw, skbb, ya_out, yb_out, skip_out):
    st = jnp.transpose(s_ref[...])
    x = x_ref[...] + _bnp(st, lbw[...], lbb[...])
    x = _mlp_res(x, mw1t[...], mb1[...], mw2t[...], mbw[...], mbb[...])
    ya_out[...] = _dot_tl(x, lpa[...])
    yb_out[...] = _dot_tl(x, lpb[...])
    ysk = _bnp(_dot(skwt[...], x), skbw[...], skbb[...])
    skip_out[...] = jnp.concatenate(
        [ysk[:, 0:_N1], ysk[:, _N0:_N0 + _N1]], axis=1)


def _stage1_head_body(sa_ref, sb_ref, lfbw, lfbb, skip_ref, gx_ref,
                      xyzs_ref, mt_ref, npbw, npbb, npw1t, npb1, npw2t,
                      nbw, nbb, mw1t, mb1, mw2t, mbw, mbb, pa_ref, pb_ref,
                      x_out, ya_out, yb_out):
    lf_full = _bnp(
        jnp.concatenate([jnp.transpose(sa_ref[...]),
                         jnp.transpose(sb_ref[...])], axis=0),
        lfbw[...], lfbb[...])
    lf = jnp.concatenate(
        [lf_full[:, 0:_N1], lf_full[:, _N0:_N0 + _N1]], axis=1)
    x = skip_ref[...] + lf
    nbr = jnp.sqrt(_spse_acc(gx_ref, xyzs_ref[...], mt_ref[...], 0.5) + 1e-12)
    h = _bnp(nbr, npbw[...], npbb[...])
    h = jax.nn.gelu(_dot(npw1t[...], h) + npb1[...])
    h = _dot(npw2t[...], h)
    x = _bnp(h, nbw[...], nbb[...]) + x
    x = _mlp_res(x, mw1t[...], mb1[...], mw2t[...], mbw[...], mbb[...])
    x_out[...] = x
    ya_out[...] = _dot_tl(x, pa_ref[...])
    yb_out[...] = _dot_tl(x, pb_ref[...])


def _bn_add_proj2_body(x_ref, sa_ref, sb_ref, bw, bb, pa_ref, pb_ref,
                       x_out, ya_out, yb_out):
    st = jnp.concatenate([jnp.transpose(sa_ref[...]),
                          jnp.transpose(sb_ref[...])], axis=0)
    x = x_ref[...] + _bnp(st, bw[...], bb[...])
    x_out[...] = x
    ya_out[...] = _dot_tl(x, pa_ref[...])
    yb_out[...] = _dot_tl(x, pb_ref[...])


def _tail1_body(x_ref, sa_ref, sb_ref, lbw, lbb, mw1t, mb1, mw2t, mbw, mbb,
                out_ref):
    st = jnp.concatenate([jnp.transpose(sa_ref[...]),
                          jnp.transpose(sb_ref[...])], axis=0)
    x = x_ref[...] + _bnp(st, lbw[...], lbb[...])
    out_ref[...] = _mlp_res(x, mw1t[...], mb1[...], mw2t[...], mbw[...],
                            mbb[...])


def _call(body, out_shapes, *args):
    return pl.pallas_call(body, out_shape=out_shapes)(*args)


def _f32(r, c):
    return jax.ShapeDtypeStruct((r, c), jnp.float32)


def _pad_cols(w, to):
    return jnp.pad(w, ((0, 0), (0, to - w.shape[1])))


def _halves(p):
    return ({'coor': p['coor'][:72], 'scale': p['scale'][:24]},
            {'coor': p['coor'][72:], 'scale': p['scale'][24:]})


def _proj_halves(w):
    return (_pad_cols(w[:, 0:96], 128), _pad_cols(w[:, 96:192], 128))


def _mlp_args(p):
    return (p['w1'].T, p['b1'][:, None], p['w2'].T,
            p['bn_w'][:, None], p['bn_b'][:, None])


def kernel(x, xyz, prev_knn, pwd, params):
    del x, prev_knn
    s0, s1, m = params['s0'], params['s1'], params['spse_m']
    blk0, blk1 = s0['blk'], s1['blk']
    bn0, bn1 = _B * _N0, _B * _N1
    xyzf = xyz.reshape(bn0, 3)
    xyzt = xyzf.T
    xyzp = jnp.pad(xyzf, ((0, 0), (0, 5)))

    knn0 = _topk16(pwd, _N0)
    idx0 = _stack_idx(knn0, _N0)
    gxyz0 = jnp.transpose(_gather_rows(xyzp, idx0)[:, :, :3], (0, 2, 1))

    x0, y0 = _call(
        _stage0_head_body, [_f32(96, bn0), _f32(bn0, 128)],
        gxyz0, xyzt, m.T,
        s0['np_bn_w'][:, None], s0['np_bn_b'][:, None], s0['np_w1'].T,
        s0['np_b1'][:, None], s0['np_w2'].T, s0['nbr_bn_w'][:, None],
        s0['nbr_bn_b'][:, None], *_mlp_args(blk0['mlp0']),
        _pad_cols(blk0['lfp0']['proj'], 128))

    gy0 = _gather_rows(y0, idx0)
    sm0 = _wmean(gy0, gxyz0, xyzt, blk0['lfp0'], 1.0, 96)
    x1, y1 = _call(
        _bn_add_proj_body, [_f32(96, bn0), _f32(bn0, 128)],
        x0, sm0, blk0['lfp0']['bn_w'][:, None], blk0['lfp0']['bn_b'][:, None],
        _pad_cols(blk0['lfp1']['proj'], 128))

    gy1 = _gather_rows(y1, idx0)
    sm1 = _wmean(gy1, gxyz0, xyzt, blk0['lfp1'], 1.0, 96)
    ya, yb, skip = _call(
        _tail0_body, [_f32(bn0, 128), _f32(bn0, 128), _f32(192, bn1)],
        x1, sm1, blk0['lfp1']['bn_w'][:, None], blk0['lfp1']['bn_b'][:, None],
        *_mlp_args(blk0['mlps0']),
        *_proj_halves(s1['lfp']['proj']), s1['skip_w'].T,
        s1['skip_bn_w'][:, None], s1['skip_bn_b'][:, None])

    lfh_a, lfh_b = _halves(s1['lfp'])
    slfa = _wmean(_gather_rows(ya, idx0), gxyz0, xyzt, lfh_a, 1.0, 96)
    slfb = _wmean(_gather_rows(yb, idx0), gxyz0, xyzt, lfh_b, 1.0, 96)

    knn1 = _topk16(pwd, _N1)
    idx1x = _stack_idx(knn1, _N0)
    idx1f = _stack_idx(knn1, _N1)
    gxyz1 = jnp.transpose(_gather_rows(xyzp, idx1x)[:, :, :3], (0, 2, 1))
    xyzst = jnp.concatenate([xyzt[:, 0:_N1], xyzt[:, _N0:_N0 + _N1]], axis=1)

    x2, y2a, y2b = _call(
        _stage1_head_body,
        [_f32(192, bn1), _f32(bn1, 128), _f32(bn1, 128)],
        slfa, slfb, s1['lfp']['bn_w'][:, None], s1['lfp']['bn_b'][:, None],
        skip, gxyz1, xyzst, m.T,
        s1['np_bn_w'][:, None], s1['np_bn_b'][:, None], s1['np_w1'].T,
        s1['np_b1'][:, None], s1['np_w2'].T, s1['nbr_bn_w'][:, None],
        s1['nbr_bn_b'][:, None], *_mlp_args(blk1['mlp0']),
        *_proj_halves(blk1['lfp0']['proj']))

    l0h_a, l0h_b = _halves(blk1['lfp0'])
    sm2a = _wmean(_gather_rows(y2a, idx1f), gxyz1, xyzst, l0h_a, 0.5, 96)
    sm2b = _wmean(_gather_rows(y2b, idx1f), gxyz1, xyzst, l0h_b, 0.5, 96)
    x3, y3a, y3b = _call(
        _bn_add_proj2_body,
        [_f32(192, bn1), _f32(bn1, 128), _f32(bn1, 128)],
        x2, sm2a, sm2b, blk1['lfp0']['bn_w'][:, None],
        blk1['lfp0']['bn_b'][:, None], *_proj_halves(blk1['lfp1']['proj']))

    l1h_a, l1h_b = _halves(blk1['lfp1'])
    sm3a = _wmean(_gather_rows(y3a, idx1f), gxyz1, xyzst, l1h_a, 0.5, 96)
    sm3b = _wmean(_gather_rows(y3b, idx1f), gxyz1, xyzst, l1h_b, 0.5, 96)
    out = _call(
        _tail1_body, [_f32(192, bn1)],
        x3, sm3a, sm3b, blk1['lfp1']['bn_w'][:, None],
        blk1['lfp1']['bn_b'][:, None], *_mlp_args(blk1['mlps0']))[0]

    return out.T.reshape(_B, _N1, 192)
